# baseline (speedup 1.0000x reference)
import functools

import jax
import jax.numpy as jnp
from jax import lax
from jax.experimental import pallas as pl
from jax.experimental.pallas import tpu as pltpu
from jax.experimental.pallas import tpu_sc as plsc

_B, _S, _H, _D = 4, 2048, 32, 128
_HD = _H * _D
_NC, _NS, _L = 2, 16, 16
_NW = _NC * _NS
_CPB = _NW // _B
_CW = _HD // _CPB
_NV = _CW // _L
_CHUNK = 64


def _lane_select_i32(vec, k):
    lanes = lax.iota(jnp.int32, _L)
    return jnp.sum(jnp.where(lanes == k, vec, 0))


def _lane_select_f32(vec, k):
    lanes = lax.iota(jnp.int32, _L)
    return jnp.sum(jnp.where(lanes == k, vec, jnp.float32(0.0)))


def _pool_sc(pooled3, meta_i, meta_f):
    mesh = plsc.VectorSubcoreMesh(core_axis_name="c", subcore_axis_name="s")

    @functools.partial(
        pl.kernel,
        mesh=mesh,
        out_type=jax.ShapeDtypeStruct((_B * _HD,), jnp.float32),
        scratch_types=[
            pltpu.VMEM((_L,), jnp.int32),
            pltpu.VMEM((_L,), jnp.float32),
            pltpu.VMEM((_CHUNK, _CW), jnp.float32),
            pltpu.VMEM((_CW,), jnp.float32),
        ],
    )
    def pool_kernel(pooled_hbm, mi_hbm, mf_hbm, x_hbm, mi_v, mf_v, buf, xv):
        wid = lax.axis_index("s") * _NC + lax.axis_index("c")
        b = wid // _CPB
        c = wid % _CPB
        col0 = pl.multiple_of(c * _CW, _CW)
        moff = pl.multiple_of(b * _L, _L)
        pltpu.sync_copy(mi_hbm.at[pl.ds(moff, _L)], mi_v)
        pltpu.sync_copy(mf_hbm.at[pl.ds(moff, _L)], mf_v)
        mi = mi_v[...]
        mf = mf_v[...]
        zero = jnp.zeros((_L,), jnp.float32)

        def segment_sum(s_idx):
            start = mi[4 * s_idx + 0]
            end = mi[4 * s_idx + 1]
            astart = mi[4 * s_idx + 2]
            nch = mi[4 * s_idx + 3]

            def chunk_body(g, acc):
                base = astart + g * _CHUNK
                cb = jnp.minimum(base, _S - _CHUNK)
                cb = pl.multiple_of(cb, 8)
                pltpu.sync_copy(
                    pooled_hbm.at[b, pl.ds(cb, _CHUNK), pl.ds(col0, _CW)], buf)
                lo = jnp.maximum(start, base) - cb
                hi = jnp.minimum(end, base + _CHUNK - 1) - cb + 1

                def row_body(r, a):
                    return tuple(
                        a[v] + buf[r, pl.ds(_L * v, _L)] for v in range(_NV))

                return lax.fori_loop(lo, hi, row_body, acc)

            return lax.fori_loop(0, nch, chunk_body,
                                 tuple(zero for _ in range(_NV)))

        acc_ctx = segment_sum(0)
        acc_q = segment_sum(1)
        sc_ctx = mf[0]
        sc_q = mf[1]
        for v in range(_NV):
            xv[pl.ds(_L * v, _L)] = acc_ctx[v] * sc_ctx + acc_q[v] * sc_q
        xoff = pl.multiple_of(b * _HD + col0, _CW)
        pltpu.sync_copy(xv, x_hbm.at[pl.ds(xoff, _CW)])

    return pool_kernel(pooled3, meta_i, meta_f)


def _mlp_body(x_ref, w1_ref, b1_ref, w2_ref, b2_ref, w3_ref, b3_ref, it_ref,
              logits_ref, zs_ref, zh_ref):
    x = x_ref[...]
    h = lax.dot_general(x, w1_ref[...], (((1,), (1,)), ((), ())),
                        precision=lax.Precision.HIGHEST,
                        preferred_element_type=jnp.float32)
    h = h + b1_ref[...]
    h = h * jax.nn.sigmoid(h)
    h = lax.dot_general(h, w2_ref[...], (((1,), (1,)), ((), ())),
                        precision=lax.Precision.HIGHEST,
                        preferred_element_type=jnp.float32)
    h = h + b2_ref[...]
    h = h * jax.nn.sigmoid(h)
    w3 = w3_ref[...]
    l0 = jnp.sum(h * w3[0:1, :], axis=1, keepdims=True) + b3_ref[0, 0]
    l1 = jnp.sum(h * w3[1:2, :], axis=1, keepdims=True) + b3_ref[0, 1]
    zs = jax.nn.sigmoid((l1 - l0) * it_ref[0, 0])
    zh = (zs > 0.5).astype(jnp.float32)
    logits_ref[...] = jnp.concatenate([l0, l1], axis=1)
    zs_ref[...] = zs
    zh_ref[...] = zh


def _mlp_tc(xr, W1, b1, W2, b2, W3, b3, inv_tau):
    n = xr.shape[0]
    return pl.pallas_call(
        _mlp_body,
        out_shape=[
            jax.ShapeDtypeStruct((n, 2), jnp.float32),
            jax.ShapeDtypeStruct((n, 1), jnp.float32),
            jax.ShapeDtypeStruct((n, 1), jnp.float32),
        ],
    )(xr, W1, b1.reshape(1, -1), W2, b2.reshape(1, -1), W3,
      b3.reshape(1, -1), inv_tau)


def kernel(pooled_input, range_ids, W1, b1, W2, b2, W3, b3, log_temp):
    pooled3 = pooled_input.reshape(_B, _S, _HD)
    r = range_ids.astype(jnp.int32)
    s0, e0, s1, e1 = r[:, 0], r[:, 1], r[:, 2], r[:, 3]
    n0 = e0 - s0 + 1
    n1 = e1 - s1 + 1
    a0 = (s0 // 8) * 8
    a1 = (s1 // 8) * 8
    k0 = (e0 - a0 + _CHUNK) // _CHUNK
    k1 = (e1 - a1 + _CHUNK) // _CHUNK
    zi = jnp.zeros((_B,), jnp.int32)
    meta_i = jnp.stack(
        [s0, e0, a0, k0, s1, e1, a1, k1] + [zi] * (_L - 8), axis=1).reshape(-1)
    zf = jnp.zeros((_B,), jnp.float32)
    meta_f = jnp.stack(
        [0.5 / n0.astype(jnp.float32),
         0.5 / n1.astype(jnp.float32)] + [zf] * (_L - 2), axis=1).reshape(-1)

    x = _pool_sc(pooled3, meta_i, meta_f)
    xr = x.reshape(_B * _H, _D)
    inv_tau = jnp.exp(-log_temp).reshape(1, 1)
    logits2, zs, zh = _mlp_tc(xr, W1, b1, W2, b2, W3, b3, inv_tau)
    logits = logits2.reshape(_B, _H, 2)
    z_soft = zs.reshape(_B, _H)
    z_hard = zh.reshape(_B, _H)
    return (z_soft, z_hard, z_hard, logits)


# speedup vs baseline: 1.1586x; 1.1586x over previous
import functools

import jax
import jax.numpy as jnp
from jax import lax
from jax.experimental import pallas as pl
from jax.experimental.pallas import tpu as pltpu
from jax.experimental.pallas import tpu_sc as plsc

_B, _S, _H, _D = 4, 2048, 32, 128
_HD = _H * _D
_NC, _NS, _L = 2, 16, 16
_NW = _NC * _NS
_CPB = _NW // _B
_CW = _HD // _CPB
_NV = _CW // _L
_CHUNK = 64


def _pool_sc(pooled3, meta_i, meta_f):
    mesh = plsc.VectorSubcoreMesh(core_axis_name="c", subcore_axis_name="s")

    @functools.partial(
        pl.kernel,
        mesh=mesh,
        out_type=jax.ShapeDtypeStruct((_B * _HD,), jnp.float32),
        scratch_types=[
            pltpu.VMEM((_L,), jnp.int32),
            pltpu.VMEM((_L,), jnp.float32),
            pltpu.VMEM((_CHUNK, _CW), jnp.float32),
            pltpu.VMEM((_CHUNK, _CW), jnp.float32),
            pltpu.VMEM((_CW,), jnp.float32),
            pltpu.SemaphoreType.DMA,
            pltpu.SemaphoreType.DMA,
        ],
    )
    def pool_kernel(pooled_hbm, mi_hbm, mf_hbm, x_hbm,
                    mi_v, mf_v, buf0, buf1, xv, sem0, sem1):
        wid = lax.axis_index("s") * _NC + lax.axis_index("c")
        b = wid // _CPB
        c = wid % _CPB
        col0 = pl.multiple_of(c * _CW, _CW)
        moff = pl.multiple_of(b * _L, _L)
        pltpu.sync_copy(mi_hbm.at[pl.ds(moff, _L)], mi_v)
        pltpu.sync_copy(mf_hbm.at[pl.ds(moff, _L)], mf_v)
        mi = mi_v[...]
        mf = mf_v[...]
        s0_, e0_, a0_, k0_ = mi[0], mi[1], mi[2], mi[3]
        s1_, e1_, a1_, k1_ = mi[4], mi[5], mi[6], mi[7]
        sc_ctx, sc_q = mf[0], mf[1]
        ktot = k0_ + k1_
        zero = jnp.zeros((_L,), jnp.float32)

        def chunk_geom(j):
            in0 = j < k0_
            st = jnp.where(in0, s0_, s1_)
            en = jnp.where(in0, e0_, e1_)
            al = jnp.where(in0, a0_, a1_)
            g = j - jnp.where(in0, 0, k0_)
            base = al + g * _CHUNK
            cb = pl.multiple_of(jnp.minimum(base, _S - _CHUNK), 8)
            lo = jnp.maximum(st, base) - cb
            hi = jnp.minimum(en, base + _CHUNK - 1) - cb + 1
            return cb, lo, hi

        def copy_desc(j, buf, sem):
            cb, _, _ = chunk_geom(jnp.minimum(j, ktot - 1))
            return pltpu.make_async_copy(
                pooled_hbm.at[b, pl.ds(cb, _CHUNK), pl.ds(col0, _CW)],
                buf, sem)

        def accum(j, buf, acc, valid):
            _, lo, hi = chunk_geom(jnp.minimum(j, ktot - 1))
            hi = jnp.where(valid, hi, lo)

            def row_body(r, a):
                return tuple(
                    a[v] + buf[r, pl.ds(_L * v, _L)] for v in range(_NV))

            acc = lax.fori_loop(lo, hi, row_body, acc)
            done0 = j == k0_ - 1

            @pl.when(done0)
            def _():
                for v in range(_NV):
                    xv[pl.ds(_L * v, _L)] = acc[v] * sc_ctx

            return tuple(jnp.where(done0, zero, a) for a in acc)

        npairs = (ktot + 1) // 2
        copy_desc(0, buf0, sem0).start()

        def pair_body(p, acc):
            c0 = 2 * p
            c1 = c0 + 1
            copy_desc(c1, buf1, sem1).start()
            copy_desc(c0, buf0, sem0).wait()
            acc = accum(c0, buf0, acc, True)

            @pl.when(p + 1 < npairs)
            def _():
                copy_desc(c0 + 2, buf0, sem0).start()

            copy_desc(c1, buf1, sem1).wait()
            return accum(c1, buf1, acc, c1 < ktot)

        acc = lax.fori_loop(0, npairs, pair_body,
                            tuple(zero for _ in range(_NV)))
        for v in range(_NV):
            xv[pl.ds(_L * v, _L)] = xv[pl.ds(_L * v, _L)] + acc[v] * sc_q
        xoff = pl.multiple_of(b * _HD + col0, _CW)
        pltpu.sync_copy(xv, x_hbm.at[pl.ds(xoff, _CW)])

    return pool_kernel(pooled3, meta_i, meta_f)


def _mlp_body(x_ref, w1_ref, b1_ref, w2_ref, b2_ref, w3_ref, b3_ref, it_ref,
              logits_ref, zs_ref, zh_ref):
    x = x_ref[...]
    h = lax.dot_general(x, w1_ref[...], (((1,), (1,)), ((), ())),
                        precision=lax.Precision.HIGHEST,
                        preferred_element_type=jnp.float32)
    h = h + b1_ref[...]
    h = h * jax.nn.sigmoid(h)
    h = lax.dot_general(h, w2_ref[...], (((1,), (1,)), ((), ())),
                        precision=lax.Precision.HIGHEST,
                        preferred_element_type=jnp.float32)
    h = h + b2_ref[...]
    h = h * jax.nn.sigmoid(h)
    w3 = w3_ref[...]
    l0 = jnp.sum(h * w3[0:1, :], axis=1, keepdims=True) + b3_ref[0, 0]
    l1 = jnp.sum(h * w3[1:2, :], axis=1, keepdims=True) + b3_ref[0, 1]
    zs = jax.nn.sigmoid((l1 - l0) * it_ref[0, 0])
    zh = (zs > 0.5).astype(jnp.float32)
    logits_ref[...] = jnp.concatenate([l0, l1], axis=1)
    zs_ref[...] = zs
    zh_ref[...] = zh


def _mlp_tc(xr, W1, b1, W2, b2, W3, b3, inv_tau):
    n = xr.shape[0]
    return pl.pallas_call(
        _mlp_body,
        out_shape=[
            jax.ShapeDtypeStruct((n, 2), jnp.float32),
            jax.ShapeDtypeStruct((n, 1), jnp.float32),
            jax.ShapeDtypeStruct((n, 1), jnp.float32),
        ],
    )(xr, W1, b1.reshape(1, -1), W2, b2.reshape(1, -1), W3,
      b3.reshape(1, -1), inv_tau)


def kernel(pooled_input, range_ids, W1, b1, W2, b2, W3, b3, log_temp):
    pooled3 = pooled_input.reshape(_B, _S, _HD)
    r = range_ids.astype(jnp.int32)
    s0, e0, s1, e1 = r[:, 0], r[:, 1], r[:, 2], r[:, 3]
    n0 = e0 - s0 + 1
    n1 = e1 - s1 + 1
    a0 = (s0 // 8) * 8
    a1 = (s1 // 8) * 8
    k0 = (e0 - a0 + _CHUNK) // _CHUNK
    k1 = (e1 - a1 + _CHUNK) // _CHUNK
    zi = jnp.zeros((_B,), jnp.int32)
    meta_i = jnp.stack(
        [s0, e0, a0, k0, s1, e1, a1, k1] + [zi] * (_L - 8), axis=1).reshape(-1)
    zf = jnp.zeros((_B,), jnp.float32)
    meta_f = jnp.stack(
        [0.5 / n0.astype(jnp.float32),
         0.5 / n1.astype(jnp.float32)] + [zf] * (_L - 2), axis=1).reshape(-1)

    x = _pool_sc(pooled3, meta_i, meta_f)
    xr = x.reshape(_B * _H, _D)
    inv_tau = jnp.exp(-log_temp).reshape(1, 1)
    logits2, zs, zh = _mlp_tc(xr, W1, b1, W2, b2, W3, b3, inv_tau)
    logits = logits2.reshape(_B, _H, 2)
    z_soft = zs.reshape(_B, _H)
    z_hard = zh.reshape(_B, _H)
    return (z_soft, z_hard, z_hard, logits)
